# baseline (speedup 1.0000x reference)
import functools

import jax
import jax.numpy as jnp
from jax import lax
from jax.experimental import pallas as pl
from jax.experimental.pallas import tpu as pltpu
from jax.experimental.pallas import tpu_sc as plsc

_LANES = 16
_NUM_CORES = 2
_NUM_SUBCORES = 16
_NW = _NUM_CORES * _NUM_SUBCORES


def _make_sc_call(N, C, H, W, L):
    HW = H * W
    BPW = N // _NW
    NV = (L + _LANES - 1) // _LANES
    NPOS = 4 * NV
    NITEM = NPOS * _LANES
    LM_PAD = NV * _LANES
    CLM = 8
    NCHUNK = (L + CLM - 1) // CLM
    ROWS = 2 * C * CLM

    mesh = plsc.VectorSubcoreMesh(
        core_axis_name="c", subcore_axis_name="s",
        num_cores=_NUM_CORES, num_subcores=_NUM_SUBCORES)

    @functools.partial(
        pl.kernel,
        out_type=jax.ShapeDtypeStruct((_NW, _LANES), jnp.float32),
        mesh=mesh,
        scratch_types=[
            pltpu.VMEM((2 * LM_PAD,), jnp.int32),
            pltpu.VMEM((HW,), jnp.int32),
            pltpu.VMEM((NITEM,), jnp.int32),
            pltpu.VMEM((NITEM,), jnp.float32),
            pltpu.VMEM((LM_PAD,), jnp.int32),
            pltpu.VMEM((6 * LM_PAD,), jnp.int32),
            pltpu.VMEM((ROWS, W), jnp.float32),
            pltpu.VMEM((ROWS, W), jnp.float32),
            pltpu.VMEM((ROWS, W), jnp.float32),
            pltpu.VMEM((ROWS, W), jnp.float32),
            pltpu.VMEM((_LANES,), jnp.float32),
            pltpu.SemaphoreType.DMA,
        ],
        compiler_params=pltpu.CompilerParams(
            needs_layout_passes=False,
            disable_bounds_checks=True,
            disable_semaphore_checks=True,
            skip_device_barrier=True,
        ),
    )
    def body(gen_hbm, tar_hbm, lm_hbm, out_hbm,
             lm_v, buf, pos_v, cm_v, y_v, row_v,
             g0, t0, g1, t1, acc_v, sem):
        cid = lax.axis_index("c")
        sid = lax.axis_index("s")
        wid = sid * _NUM_CORES + cid
        lane = lax.iota(jnp.int32, _LANES)
        dsts = ((g0, t0), (g1, t1))
        acc = jnp.zeros((_LANES,), jnp.float32)
        for b in range(BPW):
            n = wid * BPW + b
            pltpu.sync_copy(lm_hbm.at[n], lm_v)
            for k in range(NV):
                lim = min(_LANES, L - k * _LANES)
                x = jnp.clip(lm_v[pl.ds(k * _LANES, _LANES)], 1, H - 2)
                y = jnp.clip(lm_v[pl.ds(LM_PAD + k * _LANES, _LANES)],
                             1, W - 2)
                y_v[pl.ds(k * _LANES, _LANES)] = y
                lvec = lane + k * _LANES
                for ch in range(C):
                    base_row = (n * C + ch) * H
                    for rj in range(2):
                        plsc.store_scatter(
                            row_v, [lvec * (2 * C) + ch * 2 + rj],
                            base_row + (x - 1) + rj)
                valid = (lane < lim) if lim < _LANES else None
                for a in range(2):
                    for c2 in range(2):
                        v = k * 4 + a * 2 + c2
                        p = (x - 1 + a) * W + (y - 1 + c2)
                        plsc.store_scatter(buf, [p], lane + v * _LANES,
                                           mask=valid)
                        pos_v[pl.ds(v * _LANES, _LANES)] = p
            inflight = {}

            def fire(c):
                sl = pl.ds(c * ROWS, ROWS)
                dg, dt = dsts[c % 2]
                inflight[c] = (
                    pltpu.async_copy(gen_hbm.at[row_v.at[sl]], dg, sem),
                    pltpu.async_copy(tar_hbm.at[row_v.at[sl]], dt, sem),
                )

            fire(0)
            fire(1)
            for v in range(NPOS):
                lim = min(_LANES, L - (v // 4) * _LANES)
                p = pos_v[pl.ds(v * _LANES, _LANES)]
                w = plsc.load_gather(buf, [p])
                contrib = w == (lane + v * _LANES)
                if lim < _LANES:
                    contrib = jnp.logical_and(contrib, lane < lim)
                cm_v[pl.ds(v * _LANES, _LANES)] = jnp.where(
                    contrib, jnp.float32(1.0), jnp.float32(0.0))
            for c in range(NCHUNK):
                for cp in inflight.pop(c):
                    cp.wait()
                dg, dt = dsts[c % 2]
                for h in range(2):
                    q = 4 * h + (lane >> 2)
                    l = c * CLM + q
                    e = lane & 3
                    a = e >> 1
                    c2 = e & 1
                    col = plsc.load_gather(y_v, [l]) - 1 + c2
                    cm = plsc.load_gather(
                        cm_v, [((l >> 4) * 4 + e) * _LANES + (l & 15)])
                    s = jnp.zeros((_LANES,), jnp.float32)
                    for ch in range(C):
                        row = q * (2 * C) + ch * 2 + a
                        gv = plsc.load_gather(dg, [row, col])
                        tv = plsc.load_gather(dt, [row, col])
                        s = s + jnp.abs(gv - tv)
                    acc = acc + cm * s
                if c + 2 < NCHUNK:
                    fire(c + 2)
        acc_v[...] = acc
        pltpu.sync_copy(acc_v, out_hbm.at[wid])

    return body


def kernel(generated_img, target_img, lm_array, original_size):
    N, C, H, W = generated_img.shape
    L = (lm_array.shape[2] - 2) // 2
    half = ((L + _LANES - 1) // _LANES) * _LANES
    xs = lm_array[:, 0, 2:2 + L]
    ys = lm_array[:, 0, 2 + L:2 + 2 * L]
    pad = ((0, 0), (0, half - L))
    lm_pad = jnp.concatenate(
        [jnp.pad(xs, pad), jnp.pad(ys, pad)], axis=1)
    call = _make_sc_call(N, C, H, W, L)
    partials = call(generated_img.reshape(N * C * H, W),
                    target_img.reshape(N * C * H, W), lm_pad)
    return jnp.sum(partials) / jnp.float32(N * C * H * W)


# speedup vs baseline: 1.0235x; 1.0235x over previous
import functools

import jax
import jax.numpy as jnp
from jax import lax
from jax.experimental import pallas as pl
from jax.experimental.pallas import tpu as pltpu
from jax.experimental.pallas import tpu_sc as plsc

_LANES = 16
_NUM_CORES = 2
_NUM_SUBCORES = 16
_NW = _NUM_CORES * _NUM_SUBCORES


def _make_sc_call(N, C, H, W, L):
    HW = H * W
    BPW = N // _NW
    NV = (L + _LANES - 1) // _LANES
    NPOS = 4 * NV
    NITEM = NPOS * _LANES
    LM_PAD = NV * _LANES
    CLM = 8
    NCHUNK = (L + CLM - 1) // CLM
    ROWS = 2 * C * CLM

    mesh = plsc.VectorSubcoreMesh(
        core_axis_name="c", subcore_axis_name="s",
        num_cores=_NUM_CORES, num_subcores=_NUM_SUBCORES)

    @functools.partial(
        pl.kernel,
        out_type=jax.ShapeDtypeStruct((_NW, _LANES), jnp.float32),
        mesh=mesh,
        scratch_types=[
            pltpu.VMEM((1, 2 + 2 * L), jnp.int32),
            pltpu.VMEM((HW,), jnp.int32),
            pltpu.VMEM((NITEM,), jnp.int32),
            pltpu.VMEM((NITEM,), jnp.float32),
            pltpu.VMEM((LM_PAD,), jnp.int32),
            pltpu.VMEM((6 * LM_PAD,), jnp.int32),
            pltpu.VMEM((ROWS, W), jnp.float32),
            pltpu.VMEM((ROWS, W), jnp.float32),
            pltpu.VMEM((ROWS, W), jnp.float32),
            pltpu.VMEM((ROWS, W), jnp.float32),
            pltpu.VMEM((_LANES,), jnp.float32),
            pltpu.SemaphoreType.DMA,
        ],
        compiler_params=pltpu.CompilerParams(
            needs_layout_passes=False,
            disable_bounds_checks=True,
            disable_semaphore_checks=True,
            skip_device_barrier=True,
        ),
    )
    def body(gen_hbm, tar_hbm, lm_hbm, out_hbm,
             lm_v, buf, pos_v, cm_v, y_v, row_v,
             g0, t0, g1, t1, acc_v, sem):
        cid = lax.axis_index("c")
        sid = lax.axis_index("s")
        wid = sid * _NUM_CORES + cid
        lane = lax.iota(jnp.int32, _LANES)
        dsts = ((g0, t0), (g1, t1))
        acc = jnp.zeros((_LANES,), jnp.float32)
        zero16 = jnp.zeros((_LANES,), jnp.int32)
        last = jnp.int32(1 + 2 * L)
        for b in range(BPW):
            n = wid * BPW + b
            pltpu.sync_copy(lm_hbm.at[n], lm_v)
            for k in range(NV):
                lim = min(_LANES, L - k * _LANES)
                xi = jnp.minimum(lane + (2 + k * _LANES), last)
                yi = jnp.minimum(lane + (2 + L + k * _LANES), last)
                x = jnp.clip(plsc.load_gather(lm_v, [zero16, xi]), 1, H - 2)
                y = jnp.clip(plsc.load_gather(lm_v, [zero16, yi]), 1, W - 2)
                y_v[pl.ds(k * _LANES, _LANES)] = y
                lvec = lane + k * _LANES
                for ch in range(C):
                    base_row = (n * C + ch) * H
                    for rj in range(2):
                        plsc.store_scatter(
                            row_v, [lvec * (2 * C) + ch * 2 + rj],
                            base_row + (x - 1) + rj)
                valid = (lane < lim) if lim < _LANES else None
                for a in range(2):
                    for c2 in range(2):
                        v = k * 4 + a * 2 + c2
                        p = (x - 1 + a) * W + (y - 1 + c2)
                        plsc.store_scatter(buf, [p], lane + v * _LANES,
                                           mask=valid)
                        pos_v[pl.ds(v * _LANES, _LANES)] = p
            inflight = {}

            def fire(c):
                sl = pl.ds(c * ROWS, ROWS)
                dg, dt = dsts[c % 2]
                inflight[c] = (
                    pltpu.async_copy(gen_hbm.at[row_v.at[sl]], dg, sem),
                    pltpu.async_copy(tar_hbm.at[row_v.at[sl]], dt, sem),
                )

            fire(0)
            fire(1)
            for v in range(NPOS):
                lim = min(_LANES, L - (v // 4) * _LANES)
                p = pos_v[pl.ds(v * _LANES, _LANES)]
                w = plsc.load_gather(buf, [p])
                contrib = w == (lane + v * _LANES)
                if lim < _LANES:
                    contrib = jnp.logical_and(contrib, lane < lim)
                cm_v[pl.ds(v * _LANES, _LANES)] = jnp.where(
                    contrib, jnp.float32(1.0), jnp.float32(0.0))
            for c in range(NCHUNK):
                for cp in inflight.pop(c):
                    cp.wait()
                dg, dt = dsts[c % 2]
                for h in range(2):
                    q = 4 * h + (lane >> 2)
                    l = c * CLM + q
                    e = lane & 3
                    a = e >> 1
                    c2 = e & 1
                    col = plsc.load_gather(y_v, [l]) - 1 + c2
                    cm = plsc.load_gather(
                        cm_v, [((l >> 4) * 4 + e) * _LANES + (l & 15)])
                    s = jnp.zeros((_LANES,), jnp.float32)
                    for ch in range(C):
                        row = q * (2 * C) + ch * 2 + a
                        gv = plsc.load_gather(dg, [row, col])
                        tv = plsc.load_gather(dt, [row, col])
                        s = s + jnp.abs(gv - tv)
                    acc = acc + cm * s
                if c + 2 < NCHUNK:
                    fire(c + 2)
        acc_v[...] = acc * jnp.float32(1.0 / (N * C * H * W))
        pltpu.sync_copy(acc_v, out_hbm.at[wid])

    return body


def kernel(generated_img, target_img, lm_array, original_size):
    N, C, H, W = generated_img.shape
    L = (lm_array.shape[2] - 2) // 2
    call = _make_sc_call(N, C, H, W, L)
    partials = call(generated_img.reshape(N * C * H, W),
                    target_img.reshape(N * C * H, W), lm_array)
    return jnp.sum(partials)


# speedup vs baseline: 1.1367x; 1.1106x over previous
import functools

import jax
import jax.numpy as jnp
from jax import lax
from jax.experimental import pallas as pl
from jax.experimental.pallas import tpu as pltpu
from jax.experimental.pallas import tpu_sc as plsc

_LANES = 16
_NUM_CORES = 2
_NUM_SUBCORES = 16
_NW = _NUM_CORES * _NUM_SUBCORES


def _make_sc_call(N, C, H, W, L):
    HW = H * W
    BPW = N // _NW
    NV = (L + _LANES - 1) // _LANES
    NPOS = 4 * NV
    NITEM = NPOS * _LANES
    LM_PAD = NV * _LANES
    CLM = 8
    NCHUNK = (L + CLM - 1) // CLM
    ROWS = 2 * C * CLM
    CSIZE = [min(CLM, L - c * CLM) for c in range(NCHUNK)]
    G = BPW * NCHUNK

    mesh = plsc.VectorSubcoreMesh(
        core_axis_name="c", subcore_axis_name="s",
        num_cores=_NUM_CORES, num_subcores=_NUM_SUBCORES)

    @functools.partial(
        pl.kernel,
        out_type=jax.ShapeDtypeStruct((_NW, _LANES), jnp.float32),
        mesh=mesh,
        scratch_types=[
            pltpu.VMEM((1, 2 + 2 * L), jnp.int32),
            pltpu.VMEM((HW,), jnp.int32),
            pltpu.VMEM((NITEM,), jnp.int32),
            [pltpu.VMEM((NITEM,), jnp.float32)
             for _ in range(BPW)],
            [pltpu.VMEM((LM_PAD,), jnp.int32)
             for _ in range(BPW)],
            [pltpu.VMEM((6 * LM_PAD,), jnp.int32)
             for _ in range(BPW)],
            pltpu.VMEM((ROWS, W), jnp.float32),
            pltpu.VMEM((ROWS, W), jnp.float32),
            pltpu.VMEM((ROWS, W), jnp.float32),
            pltpu.VMEM((ROWS, W), jnp.float32),
            pltpu.VMEM((_LANES,), jnp.float32),
            pltpu.SemaphoreType.DMA,
        ],
        compiler_params=pltpu.CompilerParams(
            needs_layout_passes=False,
            disable_bounds_checks=True,
            disable_semaphore_checks=True,
            skip_device_barrier=True,
        ),
    )
    def body(gen_hbm, tar_hbm, lm_hbm, out_hbm,
             lm_v, buf, pos_v, cm_vs, y_vs, row_vs,
             g0, t0, g1, t1, acc_v, sem):
        cid = lax.axis_index("c")
        sid = lax.axis_index("s")
        wid = sid * _NUM_CORES + cid
        lane = lax.iota(jnp.int32, _LANES)
        dsts = ((g0, t0), (g1, t1))
        zero16 = jnp.zeros((_LANES,), jnp.int32)
        last = jnp.int32(1 + 2 * L)

        def phase1(b):
            n = wid * BPW + b
            pltpu.sync_copy(lm_hbm.at[n], lm_v)
            for k in range(NV):
                lim = min(_LANES, L - k * _LANES)
                xi = jnp.minimum(lane + (2 + k * _LANES), last)
                yi = jnp.minimum(lane + (2 + L + k * _LANES), last)
                x = jnp.clip(plsc.load_gather(lm_v, [zero16, xi]), 1, H - 2)
                y = jnp.clip(plsc.load_gather(lm_v, [zero16, yi]), 1, W - 2)
                y_vs[b][pl.ds(k * _LANES, _LANES)] = y
                lvec = lane + k * _LANES
                for ch in range(C):
                    base_row = (n * C + ch) * H
                    for rj in range(2):
                        plsc.store_scatter(
                            row_vs[b], [lvec * (2 * C) + ch * 2 + rj],
                            base_row + (x - 1) + rj)
                valid = (lane < lim) if lim < _LANES else None
                for a in range(2):
                    for c2 in range(2):
                        v = k * 4 + a * 2 + c2
                        p = (x - 1 + a) * W + (y - 1 + c2)
                        plsc.store_scatter(buf, [p], lane + v * _LANES,
                                           mask=valid)
                        pos_v[pl.ds(v * _LANES, _LANES)] = p

        def phase2(b):
            for v in range(NPOS):
                lim = min(_LANES, L - (v // 4) * _LANES)
                p = pos_v[pl.ds(v * _LANES, _LANES)]
                w = plsc.load_gather(buf, [p])
                contrib = w == (lane + v * _LANES)
                if lim < _LANES:
                    contrib = jnp.logical_and(contrib, lane < lim)
                cm_vs[b][pl.ds(v * _LANES, _LANES)] = jnp.where(
                    contrib, jnp.float32(1.0), jnp.float32(0.0))

        inflight = {}

        def fire(g):
            b, c = divmod(g, NCHUNK)
            nrows = 2 * C * CSIZE[c]
            sl = pl.ds(c * ROWS, nrows)
            dg, dt = dsts[g % 2]
            inflight[g] = (
                pltpu.async_copy(gen_hbm.at[row_vs[b].at[sl]],
                                 dg.at[pl.ds(0, nrows)], sem),
                pltpu.async_copy(tar_hbm.at[row_vs[b].at[sl]],
                                 dt.at[pl.ds(0, nrows)], sem),
            )

        def compute(g, acc):
            b, c = divmod(g, NCHUNK)
            dg, dt = dsts[g % 2]
            for h in range((4 * CSIZE[c] + _LANES - 1) // _LANES):
                q = 4 * h + (lane >> 2)
                l = c * CLM + q
                e = lane & 3
                a = e >> 1
                c2 = e & 1
                col = plsc.load_gather(y_vs[b], [l]) - 1 + c2
                cm = plsc.load_gather(
                    cm_vs[b], [((l >> 4) * 4 + e) * _LANES + (l & 15)])
                s = jnp.zeros((_LANES,), jnp.float32)
                for ch in range(C):
                    row = q * (2 * C) + ch * 2 + a
                    gv = plsc.load_gather(dg, [row, col])
                    tv = plsc.load_gather(dt, [row, col])
                    s = s + jnp.abs(gv - tv)
                acc = acc + cm * s
            return acc

        phase1(0)
        fire(0)
        fire(1)
        phase2(0)
        for b in range(1, BPW):
            phase1(b)
            phase2(b)
        acc = jnp.zeros((_LANES,), jnp.float32)
        for g in range(G):
            for cp in inflight.pop(g):
                cp.wait()
            acc = compute(g, acc)
            if g + 2 < G:
                fire(g + 2)
        acc_v[...] = acc * jnp.float32(1.0 / (N * C * H * W))
        pltpu.sync_copy(acc_v, out_hbm.at[wid])

    return body


def kernel(generated_img, target_img, lm_array, original_size):
    N, C, H, W = generated_img.shape
    L = (lm_array.shape[2] - 2) // 2
    call = _make_sc_call(N, C, H, W, L)
    partials = call(generated_img.reshape(N * C * H, W),
                    target_img.reshape(N * C * H, W), lm_array)
    return jnp.sum(partials)
